# baseline (speedup 1.0000x reference)
import functools
import jax
import jax.numpy as jnp
from jax import lax
from jax.experimental import pallas as pl
from jax.experimental.pallas import tpu as pltpu
from jax.experimental.pallas import tpu_sc as plsc

_NW = 32
_CH = 3200
_GRP = 8


def _make_sc_call(n_rows, T):
    nch = T // _CH
    n_chunk_rows = n_rows * nch
    per_w = n_chunk_rows // _NW
    n_grp = per_w // _GRP
    mesh = plsc.VectorSubcoreMesh(core_axis_name="c", subcore_axis_name="s")

    @functools.partial(
        pl.kernel,
        mesh=mesh,
        out_type=jax.ShapeDtypeStruct((n_chunk_rows, _CH), jnp.float32),
        scratch_types=[
            pltpu.VMEM((per_w,), jnp.int32),
            pltpu.VMEM((_GRP, _CH), jnp.float32),
            pltpu.VMEM((_GRP, _CH), jnp.float32),
            pltpu.SemaphoreType.DMA,
            pltpu.SemaphoreType.DMA,
            pltpu.SemaphoreType.DMA,
            pltpu.SemaphoreType.DMA,
        ],
    )
    def sc_call(in_hbm, cidx_hbm, out_hbm, idx_v, buf0, buf1, rs0, rs1, ws0, ws1):
        wid = lax.axis_index("s") * 2 + lax.axis_index("c")
        base = wid * per_w
        pltpu.sync_copy(cidx_hbm.at[pl.ds(base, per_w)], idx_v)
        bufs = (buf0, buf1)
        rsems = (rs0, rs1)
        wsems = (ws0, ws1)

        def gather(g, buf, rsem):
            return pltpu.make_async_copy(
                in_hbm.at[idx_v.at[pl.ds(g * _GRP, _GRP)]], buf, rsem
            )

        def put(g, buf, wsem):
            return pltpu.make_async_copy(
                buf, out_hbm.at[pl.ds(base + g * _GRP, _GRP)], wsem
            )

        gather(0, buf0, rs0).start()
        for g in range(n_grp):
            pb = (g + 1) % 2
            cb = g % 2
            if g + 1 < n_grp:
                if g + 1 >= 2:
                    put(g - 1, bufs[pb], wsems[pb]).wait()
                gather(g + 1, bufs[pb], rsems[pb]).start()
            gather(g, bufs[cb], rsems[cb]).wait()
            put(g, bufs[cb], wsems[cb]).start()
        put(n_grp - 2, bufs[(n_grp - 2) % 2], wsems[(n_grp - 2) % 2]).wait()
        put(n_grp - 1, bufs[(n_grp - 1) % 2], wsems[(n_grp - 1) % 2]).wait()

    return sc_call


def kernel(sources):
    S, B, C, T = sources.shape
    n_rows = S * B
    nch = T // _CH
    flat = sources.reshape(n_rows * nch, _CH)

    perm = jnp.argsort(jax.random.uniform(jax.random.key(42), (B,)))
    gidx = jnp.concatenate(
        [perm.astype(jnp.int32), (B + jnp.arange(B)).astype(jnp.int32)]
    )
    j = jnp.arange(n_rows * nch, dtype=jnp.int32)
    cidx = gidx[j // nch] * nch + (j % nch)

    out = _make_sc_call(n_rows, T)(flat, cidx)
    return out.reshape(S, B, C, T)


# speedup vs baseline: 2.9981x; 2.9981x over previous
import functools
import jax
import jax.numpy as jnp
from jax import lax
from jax.experimental import pallas as pl
from jax.experimental.pallas import tpu as pltpu
from jax.experimental.pallas import tpu_sc as plsc

_NW = 32
_CH = 32000


def _make_sc_call(n_rows, T):
    nch = T // _CH
    rows_per_w = n_rows // _NW
    n_t = rows_per_w * nch
    mesh = plsc.VectorSubcoreMesh(core_axis_name="c", subcore_axis_name="s")

    @functools.partial(
        pl.kernel,
        mesh=mesh,
        out_type=jax.ShapeDtypeStruct((n_rows, T), jnp.float32),
        scratch_types=[
            pltpu.VMEM((n_rows + 16,), jnp.int32),
            pltpu.VMEM((2, _CH), jnp.float32),
            pltpu.SemaphoreType.DMA,
            pltpu.SemaphoreType.DMA,
            pltpu.SemaphoreType.DMA,
            pltpu.SemaphoreType.DMA,
        ],
    )
    def sc_call(in_hbm, gidx_hbm, out_hbm, gidx_v, buf, rs0, rs1, ws0, ws1):
        wid = lax.axis_index("s") * 2 + lax.axis_index("c")
        pltpu.sync_copy(gidx_hbm, gidx_v)
        r0 = wid * rows_per_w
        rsems = (rs0, rs1)
        wsems = (ws0, ws1)

        def src_row(row):
            return gidx_v[pl.ds(row, 16)][0]

        def read(t, slot):
            row = r0 + t // nch
            c = t % nch
            return pltpu.make_async_copy(
                in_hbm.at[src_row(row), pl.ds(c * _CH, _CH)],
                buf.at[slot],
                rsems[slot],
            )

        def write(t, slot):
            row = r0 + t // nch
            c = t % nch
            return pltpu.make_async_copy(
                buf.at[slot],
                out_hbm.at[row, pl.ds(c * _CH, _CH)],
                wsems[slot],
            )

        read(0, 0).start()
        for t in range(n_t):
            nb = (t + 1) % 2
            cb = t % 2
            if t + 1 < n_t:
                if t + 1 >= 2:
                    write(t - 1, nb).wait()
                read(t + 1, nb).start()
            read(t, cb).wait()
            write(t, cb).start()
        write(n_t - 2, 0 if (n_t - 2) % 2 == 0 else 1).wait()
        write(n_t - 1, 0 if (n_t - 1) % 2 == 0 else 1).wait()

    return sc_call


def kernel(sources):
    S, B, C, T = sources.shape
    n_rows = S * B
    flat = sources.reshape(n_rows, T)

    perm = jnp.argsort(jax.random.uniform(jax.random.key(42), (B,)))
    gidx = jnp.concatenate(
        [perm.astype(jnp.int32), (B + jnp.arange(B)).astype(jnp.int32),
         jnp.zeros((16,), jnp.int32)]
    )

    out = _make_sc_call(n_rows, T)(flat, gidx)
    return out.reshape(S, B, C, T)
